# baseline (speedup 1.0000x reference)
import functools

import jax
import jax.numpy as jnp
from jax.experimental import pallas as pl
from jax.experimental.pallas import tpu as pltpu

_THRESHOLD = 64.0
_NEG = -1e30
_LANES = 128
_ROWS = 8


def _row_losses(x, true):
    m1 = jnp.max(x, axis=1, keepdims=True)
    eq = x == m1
    runner = jnp.max(jnp.where(eq, _NEG, x), axis=1, keepdims=True)
    cnt = jnp.sum(eq.astype(jnp.float32), axis=1, keepdims=True)
    m2 = jnp.where(cnt > 1.0, m1, runner)
    s = jnp.sum(jnp.exp(x - m1), axis=1, keepdims=True)
    masked_max = jnp.where(true == m1, m2, m1)
    margin = true - masked_max
    lse = m1 + jnp.log(s)
    l = jnp.where(margin >= 0.0, 1.0 - margin, 1.0 - true + lse)
    return jnp.maximum(l, 0.0)


def _tight_closs_kernel(xa_ref, xb_ref, true_ref, res_ref, l_ref, *, nsteps):
    p = pl.program_id(0)
    la = _row_losses(xa_ref[...], true_ref[0:_ROWS, :])
    lb = _row_losses(xb_ref[...], true_ref[_ROWS:2 * _ROWS, :])
    base = p * 2 * _ROWS
    l_ref[pl.ds(base, _ROWS), :] = la
    l_ref[pl.ds(base + _ROWS, _ROWS), :] = lb

    @pl.when(p == nsteps - 1)
    def _tail():
        l = l_ref[...]
        ones_row = jnp.ones((1, _LANES), jnp.float32)
        bc = jax.lax.dot_general(l, ones_row, (((1,), (0,)), ((), ())),
                                 precision=jax.lax.Precision.HIGHEST)
        br = bc.T
        ii = jax.lax.broadcasted_iota(jnp.int32, (_LANES, _LANES), 0)
        jj = jax.lax.broadcasted_iota(jnp.int32, (_LANES, _LANES), 1)
        prec = ((br < bc) | ((br == bc) & (jj < ii))).astype(jnp.float32)
        incl = jnp.where((br == bc) & (jj == ii), 1.0, prec)
        ones_col = jnp.ones((_LANES, 1), jnp.float32)
        rank = jax.lax.dot_general(prec, ones_col, (((1,), (0,)), ((), ())),
                                   precision=jax.lax.Precision.HIGHEST)
        csum = jax.lax.dot_general(incl, l, (((1,), (0,)), ((), ())),
                                   precision=jax.lax.Precision.HIGHEST)
        keep = (csum <= _THRESHOLD + 1.0 - rank).astype(jnp.float32)
        c1 = jnp.sum(keep * l)
        c2 = jnp.float32(_LANES) - jnp.sum(keep)
        res_ref[0, 0] = jnp.where(c1 < c2, c2, c1)


@jax.jit
def kernel(output, target):
    B, V = output.shape
    nsteps = B // (2 * _ROWS)
    rows = jnp.arange(B, dtype=jnp.int32)
    true = output[rows, target.astype(jnp.int32)].reshape(B, 1)

    res = pl.pallas_call(
        functools.partial(_tight_closs_kernel, nsteps=nsteps),
        grid=(nsteps,),
        in_specs=[
            pl.BlockSpec((_ROWS, V), lambda p: (2 * p, 0)),
            pl.BlockSpec((_ROWS, V), lambda p: (2 * p + 1, 0)),
            pl.BlockSpec((2 * _ROWS, 1), lambda p: (p, 0)),
        ],
        out_specs=pl.BlockSpec((1, 1), lambda p: (0, 0),
                               memory_space=pltpu.SMEM),
        out_shape=jax.ShapeDtypeStruct((1, 1), jnp.float32),
        scratch_shapes=[pltpu.VMEM((B, 1), jnp.float32)],
    )(output, output, true)
    return res[0, 0]
